# baseline (speedup 1.0000x reference)
import jax
from jax.experimental import pallas as pl
from jax.experimental.pallas import tpu as pltpu


_N_CHUNKS = 16
_CHUNK_ROWS = 10000 // _N_CHUNKS


def _identity_copy_kernel(x_ref, rel_ref, x_out_ref, rel_out_ref, sems, sem_r):
    copies = []
    for i in range(_N_CHUNKS):
        sl = pl.ds(i * _CHUNK_ROWS, _CHUNK_ROWS)
        copies.append(
            pltpu.make_async_copy(x_ref.at[sl, :], x_out_ref.at[sl, :], sems.at[i])
        )
    copy_r = pltpu.make_async_copy(rel_ref, rel_out_ref, sem_r)
    for c in copies:
        c.start()
    copy_r.start()
    for c in copies:
        c.wait()
    copy_r.wait()


def kernel(x, edge_index, edge_type, rel_embed):
    x_out, rel_out = pl.pallas_call(
        _identity_copy_kernel,
        in_specs=[
            pl.BlockSpec(memory_space=pl.MemorySpace.ANY),
            pl.BlockSpec(memory_space=pl.MemorySpace.ANY),
        ],
        out_specs=[
            pl.BlockSpec(memory_space=pl.MemorySpace.ANY),
            pl.BlockSpec(memory_space=pl.MemorySpace.ANY),
        ],
        out_shape=[
            jax.ShapeDtypeStruct(x.shape, x.dtype),
            jax.ShapeDtypeStruct(rel_embed.shape, rel_embed.dtype),
        ],
        scratch_shapes=[
            pltpu.SemaphoreType.DMA((_N_CHUNKS,)),
            pltpu.SemaphoreType.DMA,
        ],
    )(x, rel_embed)
    return (x_out, rel_out)


# speedup vs baseline: 15.3756x; 15.3756x over previous
import jax
from jax.experimental import pallas as pl
from jax.experimental.pallas import tpu as pltpu

_BLOCK_ROWS = 1000


def _block_copy(in_ref, out_ref):
    out_ref[...] = in_ref[...]


def kernel(x, edge_index, edge_type, rel_embed):
    n, d = x.shape
    x_out = pl.pallas_call(
        _block_copy,
        grid=(n // _BLOCK_ROWS,),
        in_specs=[pl.BlockSpec((_BLOCK_ROWS, d), lambda i: (i, 0))],
        out_specs=pl.BlockSpec((_BLOCK_ROWS, d), lambda i: (i, 0)),
        out_shape=jax.ShapeDtypeStruct(x.shape, x.dtype),
    )(x)
    rel_out = pl.pallas_call(
        _block_copy,
        out_shape=jax.ShapeDtypeStruct(rel_embed.shape, rel_embed.dtype),
    )(rel_embed)
    return (x_out, rel_out)
